# baseline (speedup 1.0000x reference)
import functools
import math

import jax
import jax.numpy as jnp
from jax import lax
from jax.experimental import pallas as pl
from jax.experimental.pallas import tpu as pltpu
from jax.experimental.pallas import tpu_sc as plsc

N = 10000
E = 320000
D = 128
L = 16
NC, NS = 2, 16
EH = E // NC
EP = EH // NS
NPAD = 10240
NTB = NPAD // NS
GW = 64
NGC = NTB // GW
DST_BITS = 14
BLK = 400

_mesh = plsc.VectorSubcoreMesh(core_axis_name="c", subcore_axis_name="s")


@functools.partial(
    pl.kernel,
    mesh=_mesh,
    compiler_params=pltpu.CompilerParams(
        needs_layout_passes=False, use_tc_tiling_on_sc=False),
    out_type=(
        jax.ShapeDtypeStruct((NC, NPAD), jnp.int32),
        jax.ShapeDtypeStruct((NC, NPAD, D), jnp.float32),
    ),
    scratch_types=[
        pltpu.VMEM((EP,), jnp.int32),
        pltpu.VMEM((EP,), jnp.int32),
        pltpu.VMEM((NPAD,), jnp.int32),
        pltpu.VMEM((NTB,), jnp.int32),
        pltpu.VMEM((NTB,), jnp.int32),
        pltpu.VMEM((GW, D), jnp.float32),
        pltpu.SemaphoreType.DMA,
        pltpu.SemaphoreType.DMA,
        pltpu.VMEM_SHARED((N, D), jnp.float32),
        pltpu.VMEM_SHARED((NS, NPAD), jnp.int32),
    ],
)
def _sc_fused(src_hbm, dst_hbm, x_hbm, m_out, hp_out,
              src_v, dst_v, win_v, m_v, d_v, rows_v, sem, fsem, x_sh, tbl_sh):
    c = lax.axis_index("c")
    s = lax.axis_index("s")

    @pl.when(s == 0)
    def _():
        pltpu.async_copy(x_hbm, x_sh, fsem)

    ebase = c * EH + s * EP
    pltpu.sync_copy(src_hbm.at[pl.ds(ebase, EP)], src_v)
    pltpu.sync_copy(dst_hbm.at[pl.ds(ebase, EP)], dst_v)

    neg1 = jnp.full((L,), -1, jnp.int32)

    def init_body(i, _):
        win_v[pl.ds(i * L, L)] = neg1
        return 0

    lax.fori_loop(0, NPAD // L, init_body, 0)

    iota = lax.broadcasted_iota(jnp.int32, (L,), 0)

    def edge_body(i, _):
        sl = pl.ds(i * L, L)
        srcv = src_v[sl]
        pv = ((i * L + iota) << DST_BITS) | dst_v[sl]
        plsc.store_scatter(win_v, [srcv], pv)
        return 0

    lax.fori_loop(0, EP // L, edge_body, 0)

    pltpu.sync_copy(win_v, tbl_sh.at[s])

    @pl.when(s == 0)
    def _():
        pltpu.make_async_copy(x_hbm, x_sh, fsem).wait()

    plsc.subcore_barrier()

    nb = s * NTB
    loads = [
        pltpu.async_copy(
            tbl_sh.at[k, pl.ds(nb, NTB)],
            win_v.at[pl.ds(k * NTB, NTB)],
            sem,
        )
        for k in range(NS)
    ]
    for cp in loads:
        cp.wait()

    def merge_body(j, _):
        m = jnp.full((L,), -1, jnp.int32)
        for k in range(NS):
            t = win_v[pl.ds(k * NTB + j * L, L)]
            m = jnp.where(t >= 0, t, m)
        m_v[pl.ds(j * L, L)] = m
        d_v[pl.ds(j * L, L)] = jnp.where(m >= 0, m & ((1 << DST_BITS) - 1), 0)
        return 0

    lax.fori_loop(0, NTB // L, merge_body, 0)

    pltpu.sync_copy(m_v, m_out.at[c, pl.ds(nb, NTB)])

    for g in range(NGC):
        pltpu.async_copy(
            x_sh.at[d_v.at[pl.ds(g * GW, GW)]],
            rows_v,
            sem,
        ).wait()
        pltpu.sync_copy(rows_v, hp_out.at[c, pl.ds(nb + g * GW, GW)])


def _dense_body(x_ref, hp0_ref, hp1_ref, m0_ref, m1_ref,
                w1_ref, b1_ref, w2_ref, b2_ref, g_ref, be_ref, o_ref):
    x = x_ref[...]
    m0 = m0_ref[...]
    m1 = m1_ref[...]
    sel = m1 >= 0
    hp = jnp.where(sel, hp1_ref[...], hp0_ref[...])
    m = jnp.where(sel, m1, m0)
    s = x + hp
    g = g_ref[...]
    be = be_ref[...]
    mu = jnp.mean(s, axis=1, keepdims=True)
    var = jnp.mean((s - mu) ** 2, axis=1, keepdims=True)
    c = (s - mu) * lax.rsqrt(var + 1e-5) * g + be
    t = lax.dot_general(c, w1_ref[...], (((1,), (1,)), ((), ())),
                        preferred_element_type=jnp.float32) + b1_ref[...]
    t = 0.5 * t * (1.0 + lax.erf(t / math.sqrt(2.0)))
    f = lax.dot_general(t, w2_ref[...], (((1,), (1,)), ((), ())),
                        preferred_element_type=jnp.float32) + b2_ref[...] + c
    mu2 = jnp.mean(f, axis=1, keepdims=True)
    var2 = jnp.mean((f - mu2) ** 2, axis=1, keepdims=True)
    h = (f - mu2) * lax.rsqrt(var2 + 1e-5) * g + be
    o_ref[...] = jnp.where(m >= 0, h, x)


def _dense(x, hp0, hp1, m0, m1, w1, b1, w2, b2, ln_g, ln_b):
    row = lambda i: (i, 0)
    rep = lambda i: (0, 0)
    return pl.pallas_call(
        _dense_body,
        grid=(N // BLK,),
        in_specs=[
            pl.BlockSpec((BLK, D), row),
            pl.BlockSpec((BLK, D), row),
            pl.BlockSpec((BLK, D), row),
            pl.BlockSpec((BLK, 1), row),
            pl.BlockSpec((BLK, 1), row),
            pl.BlockSpec((D, D), rep),
            pl.BlockSpec((1, D), rep),
            pl.BlockSpec((D, D), rep),
            pl.BlockSpec((1, D), rep),
            pl.BlockSpec((1, D), rep),
            pl.BlockSpec((1, D), rep),
        ],
        out_specs=pl.BlockSpec((BLK, D), row),
        out_shape=jax.ShapeDtypeStruct((N, D), jnp.float32),
    )(x, hp0, hp1, m0, m1, w1, b1, w2, b2, ln_g, ln_b)


def kernel(x, edge_index, w1, b1, w2, b2, ln_g, ln_b):
    m2, hp2 = _sc_fused(edge_index[0], edge_index[1], x)
    out = _dense(x, hp2[0], hp2[1], m2[0][:, None], m2[1][:, None],
                 w1, b1[None, :], w2, b2[None, :],
                 ln_g[None, :], ln_b[None, :])
    return out


# speedup vs baseline: 1.0220x; 1.0220x over previous
import functools
import math

import jax
import jax.numpy as jnp
from jax import lax
from jax.experimental import pallas as pl
from jax.experimental.pallas import tpu as pltpu
from jax.experimental.pallas import tpu_sc as plsc

N = 10000
E = 320000
D = 128
L = 16
NC, NS = 2, 16
EH = E // NC
EP = EH // NS
NPAD = 10240
NTB = NPAD // NS
GW = 32
NGC = NTB // GW
DST_BITS = 14
BLK = 400

_mesh = plsc.VectorSubcoreMesh(core_axis_name="c", subcore_axis_name="s")


@functools.partial(
    pl.kernel,
    mesh=_mesh,
    compiler_params=pltpu.CompilerParams(
        needs_layout_passes=False, use_tc_tiling_on_sc=False),
    out_type=(
        jax.ShapeDtypeStruct((NC, NPAD), jnp.int32),
        jax.ShapeDtypeStruct((NC, NPAD, D), jnp.float32),
    ),
    scratch_types=[
        pltpu.VMEM((EP,), jnp.int32),
        pltpu.VMEM((EP,), jnp.int32),
        pltpu.VMEM((NPAD,), jnp.int32),
        pltpu.VMEM((NTB,), jnp.int32),
        pltpu.VMEM((NTB,), jnp.int32),
        pltpu.VMEM((GW, D), jnp.float32),
        pltpu.VMEM((GW, D), jnp.float32),
        pltpu.SemaphoreType.DMA,
        pltpu.SemaphoreType.DMA,
        pltpu.SemaphoreType.DMA,
        pltpu.SemaphoreType.DMA,
        pltpu.VMEM_SHARED((N, D), jnp.float32),
        pltpu.VMEM_SHARED((NS, NPAD), jnp.int32),
    ],
)
def _sc_fused(src_hbm, dst_hbm, x_hbm, m_out, hp_out,
              src_v, dst_v, win_v, m_v, d_v, rows_a, rows_b,
              sem, sem2, fsem, fsem2, x_sh, tbl_sh):
    c = lax.axis_index("c")
    s = lax.axis_index("s")

    @pl.when(s == 0)
    def _():
        pltpu.async_copy(x_hbm, x_sh, fsem)

    ebase = c * EH + s * EP
    pltpu.sync_copy(src_hbm.at[pl.ds(ebase, EP)], src_v)
    pltpu.sync_copy(dst_hbm.at[pl.ds(ebase, EP)], dst_v)

    neg1 = jnp.full((L,), -1, jnp.int32)

    def init_body(i, _):
        win_v[pl.ds(i * L, L)] = neg1
        return 0

    lax.fori_loop(0, NPAD // L, init_body, 0)

    iota = lax.broadcasted_iota(jnp.int32, (L,), 0)

    def edge_body(i, _):
        sl = pl.ds(i * L, L)
        srcv = src_v[sl]
        pv = ((i * L + iota) << DST_BITS) | dst_v[sl]
        plsc.store_scatter(win_v, [srcv], pv)
        return 0

    lax.fori_loop(0, EP // L, edge_body, 0)

    pltpu.sync_copy(win_v, tbl_sh.at[s])

    @pl.when(s == 0)
    def _():
        pltpu.make_async_copy(x_hbm, x_sh, fsem).wait()

    plsc.subcore_barrier()

    nb = s * NTB
    loads = [
        pltpu.async_copy(
            tbl_sh.at[k, pl.ds(nb, NTB)],
            win_v.at[pl.ds(k * NTB, NTB)],
            sem,
        )
        for k in range(NS)
    ]
    for cp in loads:
        cp.wait()

    def merge_body(j, _):
        m = jnp.full((L,), -1, jnp.int32)
        for k in range(NS):
            t = win_v[pl.ds(k * NTB + j * L, L)]
            m = jnp.where(t >= 0, t, m)
        m_v[pl.ds(j * L, L)] = m
        d_v[pl.ds(j * L, L)] = jnp.where(m >= 0, m & ((1 << DST_BITS) - 1), 0)
        return 0

    lax.fori_loop(0, NTB // L, merge_body, 0)

    pltpu.sync_copy(m_v, m_out.at[c, pl.ds(nb, NTB)])

    bufs = [rows_a, rows_b]
    gsems = [sem, sem2]
    fsems = [fsem, fsem2]

    def _gather(g):
        return pltpu.async_copy(
            x_sh.at[d_v.at[pl.ds(g * GW, GW)]], bufs[g % 2], gsems[g % 2])

    flushes = [None, None]
    gcur = _gather(0)
    for g in range(NGC):
        gcur.wait()
        if g + 1 < NGC:
            nxt = (g + 1) % 2
            if flushes[nxt] is not None:
                flushes[nxt].wait()
                flushes[nxt] = None
            gcur = _gather(g + 1)
        flushes[g % 2] = pltpu.async_copy(
            bufs[g % 2], hp_out.at[c, pl.ds(nb + g * GW, GW)], fsems[g % 2])
    for f in flushes:
        if f is not None:
            f.wait()


def _dense_body(x_ref, hp0_ref, hp1_ref, m0_ref, m1_ref,
                w1_ref, b1_ref, w2_ref, b2_ref, g_ref, be_ref, o_ref):
    x = x_ref[...]
    m0 = m0_ref[...]
    m1 = m1_ref[...]
    sel = m1 >= 0
    hp = jnp.where(sel, hp1_ref[...], hp0_ref[...])
    m = jnp.where(sel, m1, m0)
    s = x + hp
    g = g_ref[...]
    be = be_ref[...]
    mu = jnp.mean(s, axis=1, keepdims=True)
    var = jnp.mean((s - mu) ** 2, axis=1, keepdims=True)
    c = (s - mu) * lax.rsqrt(var + 1e-5) * g + be
    t = lax.dot_general(c, w1_ref[...], (((1,), (1,)), ((), ())),
                        preferred_element_type=jnp.float32) + b1_ref[...]
    t = 0.5 * t * (1.0 + lax.erf(t / math.sqrt(2.0)))
    f = lax.dot_general(t, w2_ref[...], (((1,), (1,)), ((), ())),
                        preferred_element_type=jnp.float32) + b2_ref[...] + c
    mu2 = jnp.mean(f, axis=1, keepdims=True)
    var2 = jnp.mean((f - mu2) ** 2, axis=1, keepdims=True)
    h = (f - mu2) * lax.rsqrt(var2 + 1e-5) * g + be
    o_ref[...] = jnp.where(m >= 0, h, x)


def _dense(x, hp0, hp1, m0, m1, w1, b1, w2, b2, ln_g, ln_b):
    row = lambda i: (i, 0)
    rep = lambda i: (0, 0)
    return pl.pallas_call(
        _dense_body,
        grid=(N // BLK,),
        in_specs=[
            pl.BlockSpec((BLK, D), row),
            pl.BlockSpec((BLK, D), row),
            pl.BlockSpec((BLK, D), row),
            pl.BlockSpec((BLK, 1), row),
            pl.BlockSpec((BLK, 1), row),
            pl.BlockSpec((D, D), rep),
            pl.BlockSpec((1, D), rep),
            pl.BlockSpec((D, D), rep),
            pl.BlockSpec((1, D), rep),
            pl.BlockSpec((1, D), rep),
            pl.BlockSpec((1, D), rep),
        ],
        out_specs=pl.BlockSpec((BLK, D), row),
        out_shape=jax.ShapeDtypeStruct((N, D), jnp.float32),
    )(x, hp0, hp1, m0, m1, w1, b1, w2, b2, ln_g, ln_b)


def kernel(x, edge_index, w1, b1, w2, b2, ln_g, ln_b):
    m2, hp2 = _sc_fused(edge_index[0], edge_index[1], x)
    out = _dense(x, hp2[0], hp2[1], m2[0][:, None], m2[1][:, None],
                 w1, b1[None, :], w2, b2[None, :],
                 ln_g[None, :], ln_b[None, :])
    return out


# speedup vs baseline: 1.1388x; 1.1143x over previous
import functools
import math

import jax
import jax.numpy as jnp
from jax import lax
from jax.experimental import pallas as pl
from jax.experimental.pallas import tpu as pltpu
from jax.experimental.pallas import tpu_sc as plsc

N = 10000
E = 320000
D = 128
L = 16
NC, NS = 2, 16
NW = NC * NS
EP = E // NW
NPAD = 10240
NT = NPAD // NW
GW = 80
NGC = NT // GW
DST_BITS = 14
BLK = 400

_mesh = plsc.VectorSubcoreMesh(core_axis_name="c", subcore_axis_name="s")


def _wid():
    return lax.axis_index("s") * NC + lax.axis_index("c")


@functools.partial(
    pl.kernel,
    mesh=_mesh,
    compiler_params=pltpu.CompilerParams(needs_layout_passes=False),
    out_type=jax.ShapeDtypeStruct((NW, NPAD), jnp.int32),
    scratch_types=[
        pltpu.VMEM((EP,), jnp.int32),
        pltpu.VMEM((EP,), jnp.int32),
        pltpu.VMEM((NPAD,), jnp.int32),
        pltpu.SemaphoreType.DMA,
        pltpu.SemaphoreType.DMA,
    ],
)
def _sc_build(src_hbm, dst_hbm, tbl_hbm, src_v, dst_v, win_v, ssem, dsem):
    wid = _wid()
    base = wid * EP
    scp = pltpu.async_copy(src_hbm.at[pl.ds(base, EP)], src_v, ssem)
    dcp = pltpu.async_copy(dst_hbm.at[pl.ds(base, EP)], dst_v, dsem)

    neg1 = jnp.full((L,), -1, jnp.int32)

    def init_body(i, _):
        win_v[pl.ds(i * L, L)] = neg1
        return 0

    lax.fori_loop(0, NPAD // L, init_body, 0)
    scp.wait()
    dcp.wait()

    iota = lax.broadcasted_iota(jnp.int32, (L,), 0)

    def edge_body(i, _):
        sl = pl.ds(i * L, L)
        srcv = src_v[sl]
        pv = ((i * L + iota) << DST_BITS) | dst_v[sl]
        plsc.store_scatter(win_v, [srcv], pv)
        return 0

    lax.fori_loop(0, EP // L, edge_body, 0)
    pltpu.sync_copy(win_v, tbl_hbm.at[wid])


@functools.partial(
    pl.kernel,
    mesh=_mesh,
    compiler_params=pltpu.CompilerParams(
        needs_layout_passes=False, use_tc_tiling_on_sc=False),
    out_type=(
        jax.ShapeDtypeStruct((NPAD,), jnp.int32),
        jax.ShapeDtypeStruct((NPAD, D), jnp.float32),
    ),
    scratch_types=[
        pltpu.VMEM((NW, NT), jnp.int32),
        pltpu.VMEM((NT,), jnp.int32),
        pltpu.VMEM((NT,), jnp.int32),
        pltpu.VMEM((NT // 2, D), jnp.float32),
        pltpu.SemaphoreType.DMA,
        pltpu.VMEM_SHARED((N, D), jnp.float32),
        pltpu.SemaphoreType.DMA,
    ],
)
def _sc_merge_gather(tbl_hbm, x_hbm, m_out, hp_out, wa_v, m_v, d_v, rows_v, sem,
                     x_sh, fsem):
    def scoped():
        wid = _wid()
        nbase = wid * NT

        @pl.when(lax.axis_index("s") == 0)
        def _():
            pltpu.async_copy(x_hbm, x_sh, fsem)

        pltpu.sync_copy(tbl_hbm.at[:, pl.ds(nbase, NT)], wa_v)

        def merge_body(j, _):
            sl = pl.ds(j * L, L)
            m = jnp.full((L,), -1, jnp.int32)
            for k in range(NW):
                t = wa_v[k, sl]
                m = jnp.where(t >= 0, t, m)
            m_v[sl] = m
            d_v[sl] = jnp.where(m >= 0, m & ((1 << DST_BITS) - 1), 0)
            return 0

        lax.fori_loop(0, NT // L, merge_body, 0)

        @pl.when(lax.axis_index("s") == 0)
        def _():
            pltpu.make_async_copy(x_hbm, x_sh, fsem).wait()

        plsc.subcore_barrier()

        pltpu.sync_copy(m_v, m_out.at[pl.ds(nbase, NT)])
        for h in range(2):
            copies = [
                pltpu.async_copy(
                    x_sh.at[d_v.at[pl.ds(h * (NT // 2) + g * GW, GW)]],
                    rows_v.at[pl.ds(g * GW, GW)],
                    sem,
                )
                for g in range(NGC // 2)
            ]
            for cp in copies:
                cp.wait()
            pltpu.sync_copy(rows_v, hp_out.at[pl.ds(nbase + h * (NT // 2), NT // 2)])

    scoped()


def _dense_body(x_ref, hp_ref, m_ref, w1_ref, b1_ref, w2_ref, b2_ref,
                g_ref, be_ref, o_ref):
    x = x_ref[...]
    s = x + hp_ref[...]
    g = g_ref[...]
    be = be_ref[...]
    mu = jnp.mean(s, axis=1, keepdims=True)
    var = jnp.mean((s - mu) ** 2, axis=1, keepdims=True)
    c = (s - mu) * lax.rsqrt(var + 1e-5) * g + be
    t = lax.dot_general(c, w1_ref[...], (((1,), (1,)), ((), ())),
                        preferred_element_type=jnp.float32) + b1_ref[...]
    t = 0.5 * t * (1.0 + lax.erf(t / math.sqrt(2.0)))
    f = lax.dot_general(t, w2_ref[...], (((1,), (1,)), ((), ())),
                        preferred_element_type=jnp.float32) + b2_ref[...] + c
    mu2 = jnp.mean(f, axis=1, keepdims=True)
    var2 = jnp.mean((f - mu2) ** 2, axis=1, keepdims=True)
    h = (f - mu2) * lax.rsqrt(var2 + 1e-5) * g + be
    o_ref[...] = jnp.where(m_ref[...] >= 0, h, x)


def _dense(x, hp, m, w1, b1, w2, b2, ln_g, ln_b):
    row = lambda i: (i, 0)
    rep = lambda i: (0, 0)
    return pl.pallas_call(
        _dense_body,
        grid=(N // BLK,),
        in_specs=[
            pl.BlockSpec((BLK, D), row),
            pl.BlockSpec((BLK, D), row),
            pl.BlockSpec((BLK, 1), row),
            pl.BlockSpec((D, D), rep),
            pl.BlockSpec((1, D), rep),
            pl.BlockSpec((D, D), rep),
            pl.BlockSpec((1, D), rep),
            pl.BlockSpec((1, D), rep),
            pl.BlockSpec((1, D), rep),
        ],
        out_specs=pl.BlockSpec((BLK, D), row),
        out_shape=jax.ShapeDtypeStruct((N, D), jnp.float32),
    )(x, hp, m, w1, b1, w2, b2, ln_g, ln_b)


def kernel(x, edge_index, w1, b1, w2, b2, ln_g, ln_b):
    tbl = _sc_build(edge_index[0], edge_index[1])
    m, hp = _sc_merge_gather(tbl, x)
    out = _dense(x, hp, m[:, None], w1, b1[None, :], w2, b2[None, :],
                 ln_g[None, :], ln_b[None, :])
    return out
